# baseline (speedup 1.0000x reference)
import functools

import jax
import jax.numpy as jnp
from jax import lax
from jax.experimental import pallas as pl
from jax.experimental.pallas import tpu as pltpu
from jax.experimental.pallas import tpu_sc as plsc

H = 64
SUB = 128
KS = (9, 70, 200, 50)
NB = 4


@functools.lru_cache(maxsize=None)
def _make_feature_call(batch_size, k, vocab):
    info = plsc.get_sparse_core_info()
    nc, ns = info.num_cores, info.num_subcores
    nw = nc * ns
    assert batch_size == nw * SUB

    mesh = plsc.VectorSubcoreMesh(core_axis_name="c", subcore_axis_name="s")

    @functools.partial(
        pl.kernel,
        mesh=mesh,
        out_type=jax.ShapeDtypeStruct((batch_size * k, H), jnp.float32),
        scratch_types=[
            pltpu.VMEM((k, SUB), jnp.int32),
            pltpu.VMEM((NB, SUB, H), jnp.float32),
            pltpu.SemaphoreType.DMA,
            pltpu.SemaphoreType.DMA((NB,)),
            pltpu.SemaphoreType.DMA((NB,)),
        ],
        compiler_params=pltpu.CompilerParams(use_tc_tiling_on_sc=False),
    )
    def embed(idx_w, tbl, out_hbm, idx_v, rows, isem, gsem, ssem):
        wid = lax.axis_index("s") * nc + lax.axis_index("c")
        base = wid * SUB * k
        pltpu.async_copy(idx_w.at[wid], idx_v, isem).wait()

        ngrp = (k + NB - 1) // NB

        def grp(g, carry):
            for b in range(NB):
                s = g * NB + b

                @pl.when(jnp.logical_and(s < k, s >= NB))
                def _(b=b):
                    pltpu.make_async_copy(
                        rows.at[b], out_hbm.at[pl.ds(0, SUB)],
                        ssem.at[b]).wait()

                @pl.when(s < k)
                def _(b=b, s=s):
                    pltpu.async_copy(
                        tbl.at[idx_v.at[s]], rows.at[b], gsem.at[b])
            for b in range(NB):
                s = g * NB + b

                @pl.when(s < k)
                def _(b=b, s=s):
                    pltpu.make_async_copy(
                        tbl.at[pl.ds(0, SUB)], rows.at[b], gsem.at[b]).wait()
                    pltpu.async_copy(
                        rows.at[b], out_hbm.at[pl.ds(base + s * SUB, SUB)],
                        ssem.at[b])
            return carry

        lax.fori_loop(0, ngrp, grp, 0)
        for b in range(NB):
            pltpu.make_async_copy(
                rows.at[b], out_hbm.at[pl.ds(0, SUB)], ssem.at[b]).wait()

    return embed


def kernel(demographics_ints, vital_signs_ints, dx_ints, proc_ints,
           demo_table, vital_table, dx_table, proc_table, visit_table):
    batch_size = demographics_ints.shape[0]
    outs = []
    for ints, tbl, k in zip(
            (demographics_ints, vital_signs_ints, dx_ints, proc_ints),
            (demo_table, vital_table, dx_table, proc_table), KS):
        embed = _make_feature_call(batch_size, k, tbl.shape[0])
        flat = embed(ints.astype(jnp.int32).reshape(batch_size // SUB, k, SUB), tbl)
        outs.append(flat.reshape(batch_size, k, H))
    demo_emb, vital_emb, dx_emb, proc_emb = outs
    visit_emb = jnp.broadcast_to(visit_table[None, :, :],
                                 (batch_size, 1, visit_table.shape[1]))
    mask_visit = jnp.ones((batch_size, 1), dtype=jnp.float32)
    mask_demo = jnp.ones((batch_size, KS[0]), dtype=jnp.float32)
    mask_vital = jnp.ones((batch_size, KS[1]), dtype=jnp.float32)
    return (demo_emb, vital_emb, dx_emb, proc_emb, visit_emb,
            mask_visit, mask_demo, mask_vital)
